# baseline (speedup 1.0000x reference)
import functools

import jax
import jax.numpy as jnp
from jax import lax
from jax.experimental import pallas as pl
from jax.experimental.pallas import tpu as pltpu
from jax.experimental.pallas import tpu_sc as plsc

_NB = 53
_ND = 50
_SIDE = 4
_BATCH = 16384
_FACT50 = (125000, 2500, 50, 1)

_NW = 32
_BPW = _BATCH // _NW
_GROUPS = _BPW // 16


def _box_mm(b_ref, o_ref):
    i = lax.broadcasted_iota(jnp.int32, (_NB, _ND), 0)
    a = lax.broadcasted_iota(jnp.int32, (_NB, _ND), 1)
    band = ((i - a >= 0) & (i - a <= 3)).astype(jnp.float32)
    o_ref[0] = jnp.dot(b_ref[0], band, preferred_element_type=jnp.float32)


def _box_mid2(a_ref, o_ref):
    v = a_ref[...]
    s = v[:, :, 0:50, :] + v[:, :, 1:51, :] + v[:, :, 2:52, :] + v[:, :, 3:53, :]
    o_ref[...] = s[:, 0:50] + s[:, 1:51] + s[:, 2:52] + s[:, 3:53]


def _box_lead(a_ref, o_ref):
    v = a_ref[...]
    o_ref[...] = v[0:50] + v[1:51] + v[2:52] + v[3:53]


def _build_table(b_m):
    b2d = b_m.reshape(_NB, _NB * _NB, _NB)
    a = pl.pallas_call(
        _box_mm,
        grid=(_NB,),
        in_specs=[pl.BlockSpec((1, _NB * _NB, _NB), lambda i: (i, 0, 0))],
        out_specs=pl.BlockSpec((1, _NB * _NB, _ND), lambda i: (i, 0, 0)),
        out_shape=jax.ShapeDtypeStruct((_NB, _NB * _NB, _ND), jnp.float32),
    )(b2d)
    a4 = a.reshape(_NB, _NB, _NB, _ND)
    b = pl.pallas_call(
        _box_mid2,
        grid=(_NB,),
        in_specs=[pl.BlockSpec((1, _NB, _NB, _ND), lambda i: (i, 0, 0, 0))],
        out_specs=pl.BlockSpec((1, _ND, _ND, _ND), lambda i: (i, 0, 0, 0)),
        out_shape=jax.ShapeDtypeStruct((_NB, _ND, _ND, _ND), jnp.float32),
    )(a4)
    b2 = b.reshape(_NB, 1000, 125)
    c = 200
    t = pl.pallas_call(
        _box_lead,
        grid=(1000 // c,),
        in_specs=[pl.BlockSpec((_NB, c, 125), lambda i: (0, i, 0))],
        out_specs=pl.BlockSpec((_ND, c, 125), lambda i: (0, i, 0)),
        out_shape=jax.ShapeDtypeStruct((_ND, 1000, 125), jnp.float32),
    )(b2)
    return t.reshape(_ND ** 4)


def _sc_lookup(x_flat, table2d, dw, mn):
    mesh = plsc.VectorSubcoreMesh(core_axis_name="c", subcore_axis_name="s")

    @functools.partial(
        pl.kernel,
        mesh=mesh,
        out_type=jax.ShapeDtypeStruct((_BATCH,), jnp.float32),
        scratch_types=[
            pltpu.VMEM((_BPW * 4,), jnp.float32),
            pltpu.VMEM((16,), jnp.float32),
            pltpu.VMEM((16,), jnp.float32),
            pltpu.VMEM((128,), jnp.int32),
            pltpu.VMEM((128,), jnp.int32),
            pltpu.VMEM((128,), jnp.int32),
            pltpu.VMEM((128,), jnp.int32),
            pltpu.VMEM((_BPW,), jnp.float32),
            pltpu.SemaphoreType.DMA,
        ],
    )
    def body(x_hbm, t_hbm, dw_hbm, mn_hbm, out_hbm,
             xbuf, dwv, mnv, r0, r1, r2, r3, ybuf, sem):
        wid = lax.axis_index("s") * 2 + lax.axis_index("c")
        base = wid * _BPW
        for d in range(4):
            pltpu.sync_copy(x_hbm.at[pl.ds(d * _BATCH + base, _BPW)],
                            xbuf.at[pl.ds(d * _BPW, _BPW)])
        pltpu.sync_copy(dw_hbm, dwv.at[pl.ds(0, 4)])
        pltpu.sync_copy(mn_hbm, mnv.at[pl.ds(0, 4)])
        rowrefs = (r0, r1, r2, r3)
        dwvec = dwv[...]
        mnvec = mnv[...]
        for g in range(_GROUPS):
            f = jnp.zeros((16,), jnp.int32)
            for d in range(4):
                xv = xbuf[pl.ds(d * _BPW + g * 16, 16)]
                a = ((xv - mnvec[d]) / dwvec[d]).astype(jnp.int32)
                f = f + a * _FACT50[d]
            j, o = divmod(g, 8)
            rowrefs[j][pl.ds(o * 16, 16)] = f
        for j in range(4):
            pltpu.async_copy(t_hbm.at[rowrefs[j]],
                             ybuf.at[pl.ds(j * 128, 128)], sem).wait()
        pltpu.sync_copy(ybuf, out_hbm.at[pl.ds(base, _BPW)])

    return body(x_flat, table2d, dw, mn)


def kernel(x, b_m, div_widths, min_dim_ranges, dim_order, dim_factors,
           bump_ind_offsets):
    table2d = _build_table(b_m)
    y = _sc_lookup(x.T.reshape(-1), table2d, div_widths, min_dim_ranges)
    return y.reshape(_BATCH, 1)


# speedup vs baseline: 2.9224x; 2.9224x over previous
import functools

import jax
import jax.numpy as jnp
from jax import lax
from jax.experimental import pallas as pl
from jax.experimental.pallas import tpu as pltpu
from jax.experimental.pallas import tpu_sc as plsc

_NB = 53
_BATCH = 16384
_N = _NB ** 4
_FACT = (_NB ** 3, _NB ** 2, _NB, 1)

_BLK = 4096
_GRID = 16
_ROWS = (_GRID + 1) * _BLK

_NW = 32
_BPW = _BATCH // _NW
_GROUPS = _BPW // 16


def _shifted(v, t, out_rows):
    dr, dl = divmod(t, 128)
    if dl == 0:
        return v[dr:dr + out_rows]
    w = pltpu.roll(v, 128 - dl, 1)
    a = w[dr:dr + out_rows]
    b = w[dr + 1:dr + 1 + out_rows]
    lanes = lax.broadcasted_iota(jnp.int32, (out_rows, 128), 1)
    return jnp.where(lanes < 128 - dl, a, b)


def _box4(v, s, out_rows):
    mid = out_rows + (2 * s) // 128 + 1
    x = v[0:mid] + _shifted(v, s, mid)
    return x[0:out_rows] + _shifted(x, 2 * s, out_rows)


def _fused_box(a_ref, b_ref, o_ref):
    u = jnp.concatenate([a_ref[...], b_ref[...]], axis=0)
    l3 = _BLK + (3 * _FACT[0]) // 128 + 3
    l2 = l3 + (3 * _FACT[1]) // 128 + 3
    l1 = l2 + (3 * _FACT[2]) // 128 + 3
    x = _box4(u, 1, l1)
    x = _box4(x, _FACT[2], l2)
    x = _box4(x, _FACT[1], l3)
    o_ref[...] = _box4(x, _FACT[0], _BLK)


def _build_table(b_m):
    bp = jnp.pad(b_m, (0, _ROWS * 128 - _N)).reshape(_ROWS, 128)
    t = pl.pallas_call(
        _fused_box,
        grid=(_GRID,),
        in_specs=[
            pl.BlockSpec((_BLK, 128), lambda i: (i, 0)),
            pl.BlockSpec((_BLK, 128), lambda i: (i + 1, 0)),
        ],
        out_specs=pl.BlockSpec((_BLK, 128), lambda i: (i, 0)),
        out_shape=jax.ShapeDtypeStruct((_ROWS, 128), jnp.float32),
    )(bp, bp)
    return t.reshape(_ROWS * 128)


def _sc_lookup(x_flat, table, dw, mn):
    mesh = plsc.VectorSubcoreMesh(core_axis_name="c", subcore_axis_name="s")

    @functools.partial(
        pl.kernel,
        mesh=mesh,
        out_type=jax.ShapeDtypeStruct((_BATCH,), jnp.float32),
        scratch_types=[
            pltpu.VMEM((_BPW * 4,), jnp.float32),
            pltpu.VMEM((16,), jnp.float32),
            pltpu.VMEM((16,), jnp.float32),
            pltpu.VMEM((128,), jnp.int32),
            pltpu.VMEM((128,), jnp.int32),
            pltpu.VMEM((128,), jnp.int32),
            pltpu.VMEM((128,), jnp.int32),
            pltpu.VMEM((_BPW,), jnp.float32),
            pltpu.SemaphoreType.DMA,
        ],
    )
    def body(x_hbm, t_hbm, dw_hbm, mn_hbm, out_hbm,
             xbuf, dwv, mnv, r0, r1, r2, r3, ybuf, sem):
        wid = lax.axis_index("s") * 2 + lax.axis_index("c")
        base = wid * _BPW
        for d in range(4):
            pltpu.sync_copy(x_hbm.at[pl.ds(d * _BATCH + base, _BPW)],
                            xbuf.at[pl.ds(d * _BPW, _BPW)])
        pltpu.sync_copy(dw_hbm, dwv.at[pl.ds(0, 4)])
        pltpu.sync_copy(mn_hbm, mnv.at[pl.ds(0, 4)])
        rowrefs = (r0, r1, r2, r3)
        dwvec = dwv[...]
        mnvec = mnv[...]
        for g in range(_GROUPS):
            f = jnp.zeros((16,), jnp.int32)
            for d in range(4):
                xv = xbuf[pl.ds(d * _BPW + g * 16, 16)]
                a = ((xv - mnvec[d]) / dwvec[d]).astype(jnp.int32)
                f = f + a * _FACT[d]
            j, o = divmod(g, 8)
            rowrefs[j][pl.ds(o * 16, 16)] = f
        for j in range(4):
            pltpu.async_copy(t_hbm.at[rowrefs[j]],
                             ybuf.at[pl.ds(j * 128, 128)], sem).wait()
        pltpu.sync_copy(ybuf, out_hbm.at[pl.ds(base, _BPW)])

    return body(x_flat, table, dw, mn)


def kernel(x, b_m, div_widths, min_dim_ranges, dim_order, dim_factors,
           bump_ind_offsets):
    table = _build_table(b_m)
    y = _sc_lookup(x.T.reshape(-1), table, div_widths, min_dim_ranges)
    return y.reshape(_BATCH, 1)


# speedup vs baseline: 4.1832x; 1.4314x over previous
import functools

import jax
import jax.numpy as jnp
from jax import lax
from jax.experimental import pallas as pl
from jax.experimental.pallas import tpu as pltpu
from jax.experimental.pallas import tpu_sc as plsc

_NB = 53
_BATCH = 16384
_N = _NB ** 4
_FACT = (_NB ** 3, _NB ** 2, _NB, 1)

_BLK = 4096
_GRID = 15
_ROWS = (_GRID + 1) * _BLK

_NW = 32
_BPW = _BATCH // _NW
_GROUPS = _BPW // 16


def _shifted(v, t, out_rows):
    dr, dl = divmod(t, 128)
    if dl == 0:
        return v[dr:dr + out_rows]
    w = pltpu.roll(v, 128 - dl, 1)
    a = w[dr:dr + out_rows]
    b = w[dr + 1:dr + 1 + out_rows]
    lanes = lax.broadcasted_iota(jnp.int32, (out_rows, 128), 1)
    return jnp.where(lanes < 128 - dl, a, b)


def _box4(v, s, out_rows):
    mid = out_rows + (2 * s) // 128 + 1
    x = v[0:mid] + _shifted(v, s, mid)
    return x[0:out_rows] + _shifted(x, 2 * s, out_rows)


def _fused_box(a_ref, b_ref, o_ref):
    u = jnp.concatenate([a_ref[...], b_ref[...]], axis=0)
    l1 = _BLK + (3 * _FACT[1]) // 128 + 3 + 5
    l2 = _BLK + (3 * _FACT[2]) // 128 + 3 + 2
    l3 = _BLK + 3
    x = _box4(u, _FACT[0], l1)
    x = _box4(x, _FACT[1], l2)
    x = _box4(x, _FACT[2], l3)
    o_ref[...] = _box4(x, 1, _BLK)


def _build_table(b_m):
    bp = jnp.pad(b_m, (0, _ROWS * 128 - _N)).reshape(_ROWS, 128)
    t = pl.pallas_call(
        _fused_box,
        grid=(_GRID,),
        in_specs=[
            pl.BlockSpec((_BLK, 128), lambda i: (i, 0)),
            pl.BlockSpec((_BLK, 128), lambda i: (i + 1, 0)),
        ],
        out_specs=pl.BlockSpec((_BLK, 128), lambda i: (i, 0)),
        out_shape=jax.ShapeDtypeStruct((_ROWS, 128), jnp.float32),
    )(bp, bp)
    return t.reshape(_ROWS * 128)


def _sc_lookup(x_flat, table, dw, mn):
    mesh = plsc.VectorSubcoreMesh(core_axis_name="c", subcore_axis_name="s")

    @functools.partial(
        pl.kernel,
        mesh=mesh,
        out_type=jax.ShapeDtypeStruct((_BATCH,), jnp.float32),
        scratch_types=[
            pltpu.VMEM((_BPW * 4,), jnp.float32),
            pltpu.VMEM((16,), jnp.float32),
            pltpu.VMEM((16,), jnp.float32),
            pltpu.VMEM((128,), jnp.int32),
            pltpu.VMEM((128,), jnp.int32),
            pltpu.VMEM((128,), jnp.int32),
            pltpu.VMEM((128,), jnp.int32),
            pltpu.VMEM((_BPW,), jnp.float32),
            pltpu.SemaphoreType.DMA,
        ],
    )
    def body(x_hbm, t_hbm, dw_hbm, mn_hbm, out_hbm,
             xbuf, dwv, mnv, r0, r1, r2, r3, ybuf, sem):
        wid = lax.axis_index("s") * 2 + lax.axis_index("c")
        base = wid * _BPW
        for d in range(4):
            pltpu.sync_copy(x_hbm.at[pl.ds(d * _BATCH + base, _BPW)],
                            xbuf.at[pl.ds(d * _BPW, _BPW)])
        pltpu.sync_copy(dw_hbm, dwv.at[pl.ds(0, 4)])
        pltpu.sync_copy(mn_hbm, mnv.at[pl.ds(0, 4)])
        rowrefs = (r0, r1, r2, r3)
        dwvec = dwv[...]
        mnvec = mnv[...]
        for g in range(_GROUPS):
            f = jnp.zeros((16,), jnp.int32)
            for d in range(4):
                xv = xbuf[pl.ds(d * _BPW + g * 16, 16)]
                a = ((xv - mnvec[d]) / dwvec[d]).astype(jnp.int32)
                f = f + a * _FACT[d]
            j, o = divmod(g, 8)
            rowrefs[j][pl.ds(o * 16, 16)] = f
        for j in range(4):
            pltpu.async_copy(t_hbm.at[rowrefs[j]],
                             ybuf.at[pl.ds(j * 128, 128)], sem).wait()
        pltpu.sync_copy(ybuf, out_hbm.at[pl.ds(base, _BPW)])

    return body(x_flat, table, dw, mn)


def kernel(x, b_m, div_widths, min_dim_ranges, dim_order, dim_factors,
           bump_ind_offsets):
    table = _build_table(b_m)
    y = _sc_lookup(x.T.reshape(-1), table, div_widths, min_dim_ranges)
    return y.reshape(_BATCH, 1)


# speedup vs baseline: 4.2359x; 1.0126x over previous
import functools

import jax
import jax.numpy as jnp
from jax import lax
from jax.experimental import pallas as pl
from jax.experimental.pallas import tpu as pltpu
from jax.experimental.pallas import tpu_sc as plsc

_NB = 53
_BATCH = 16384
_N = _NB ** 4
_FACT = (_NB ** 3, _NB ** 2, _NB, 1)

_BLK = 4096
_GRID = 15
_ROWS = (_GRID + 1) * _BLK

_NW = 32
_BPW = _BATCH // _NW
_GROUPS = _BPW // 16


def _shifted(v, t, out_rows):
    dr, dl = divmod(t, 128)
    if dl == 0:
        return v[dr:dr + out_rows]
    w = pltpu.roll(v, 128 - dl, 1)
    a = w[dr:dr + out_rows]
    b = w[dr + 1:dr + 1 + out_rows]
    lanes = lax.broadcasted_iota(jnp.int32, (out_rows, 128), 1)
    return jnp.where(lanes < 128 - dl, a, b)


def _box4(v, s, out_rows):
    mid = out_rows + (2 * s) // 128 + 1
    x = v[0:mid] + _shifted(v, s, mid)
    return x[0:out_rows] + _shifted(x, 2 * s, out_rows)


def _fused_box(a_ref, b_ref, o_ref):
    u = jnp.concatenate([a_ref[...], b_ref[...]], axis=0)
    l1 = _BLK + (3 * _FACT[1]) // 128 + 3 + 5
    l2 = _BLK + (3 * _FACT[2]) // 128 + 3 + 2
    l3 = _BLK + 3
    x = _box4(u, _FACT[0], l1)
    x = _box4(x, _FACT[1], l2)
    x = _box4(x, _FACT[2], l3)
    o_ref[...] = _box4(x, 1, _BLK)


def _build_table(b_m):
    bp = jnp.pad(b_m, (0, _ROWS * 128 - _N)).reshape(_ROWS, 128)
    t = pl.pallas_call(
        _fused_box,
        grid=(_GRID,),
        in_specs=[
            pl.BlockSpec((_BLK, 128), lambda i: (i, 0)),
            pl.BlockSpec((_BLK, 128), lambda i: (i + 1, 0)),
        ],
        out_specs=pl.BlockSpec((_BLK, 128), lambda i: (i, 0)),
        out_shape=jax.ShapeDtypeStruct((_ROWS, 128), jnp.float32),
    )(bp, bp)
    return t.reshape(_ROWS * 128)


def _sc_lookup(x_flat, table, dw, mn):
    mesh = plsc.VectorSubcoreMesh(core_axis_name="c", subcore_axis_name="s")

    @functools.partial(
        pl.kernel,
        mesh=mesh,
        out_type=jax.ShapeDtypeStruct((_BATCH,), jnp.float32),
        scratch_types=[
            pltpu.VMEM((_BPW * 4,), jnp.float32),
            pltpu.VMEM((16,), jnp.float32),
            pltpu.VMEM((16,), jnp.float32),
            pltpu.VMEM((128,), jnp.int32),
            pltpu.VMEM((128,), jnp.int32),
            pltpu.VMEM((128,), jnp.int32),
            pltpu.VMEM((128,), jnp.int32),
            pltpu.VMEM((_BPW,), jnp.float32),
            pltpu.SemaphoreType.DMA,
        ],
    )
    def body(x_hbm, t_hbm, dw_hbm, mn_hbm, out_hbm,
             xbuf, dwv, mnv, r0, r1, r2, r3, ybuf, sem):
        wid = lax.axis_index("s") * 2 + lax.axis_index("c")
        base = wid * _BPW
        for d in range(4):
            pltpu.sync_copy(x_hbm.at[pl.ds(d * _BATCH + base, _BPW)],
                            xbuf.at[pl.ds(d * _BPW, _BPW)])
        pltpu.sync_copy(dw_hbm, dwv.at[pl.ds(0, 4)])
        pltpu.sync_copy(mn_hbm, mnv.at[pl.ds(0, 4)])
        rowrefs = (r0, r1, r2, r3)
        dwvec = dwv[...]
        mnvec = mnv[...]
        for g in range(_GROUPS):
            f = jnp.zeros((16,), jnp.int32)
            for d in range(4):
                xv = xbuf[pl.ds(d * _BPW + g * 16, 16)]
                a = ((xv - mnvec[d]) / dwvec[d]).astype(jnp.int32)
                f = f + a * _FACT[d]
            j, o = divmod(g, 8)
            rowrefs[j][pl.ds(o * 16, 16)] = f
        copies = [pltpu.async_copy(t_hbm.at[rowrefs[j]],
                                   ybuf.at[pl.ds(j * 128, 128)], sem)
                  for j in range(4)]
        for c in copies:
            c.wait()
        pltpu.sync_copy(ybuf, out_hbm.at[pl.ds(base, _BPW)])

    return body(x_flat, table, dw, mn)


def kernel(x, b_m, div_widths, min_dim_ranges, dim_order, dim_factors,
           bump_ind_offsets):
    table = _build_table(b_m)
    y = _sc_lookup(x.T.reshape(-1), table, div_widths, min_dim_ranges)
    return y.reshape(_BATCH, 1)


# speedup vs baseline: 4.3207x; 1.0200x over previous
import functools

import jax
import jax.numpy as jnp
from jax import lax
from jax.experimental import pallas as pl
from jax.experimental.pallas import tpu as pltpu
from jax.experimental.pallas import tpu_sc as plsc

_NB = 53
_BATCH = 16384
_N = _NB ** 4
_FACT = (_NB ** 3, _NB ** 2, _NB, 1)

_BLK = 4096
_GRID = 15
_ROWS = (_GRID + 1) * _BLK

_NW = 32
_BPW = _BATCH // _NW
_GROUPS = _BPW // 16


def _shifted(v, t, out_rows):
    dr, dl = divmod(t, 128)
    if dl == 0:
        return v[dr:dr + out_rows]
    w = pltpu.roll(v, 128 - dl, 1)
    a = w[dr:dr + out_rows]
    b = w[dr + 1:dr + 1 + out_rows]
    lanes = lax.broadcasted_iota(jnp.int32, (out_rows, 128), 1)
    return jnp.where(lanes < 128 - dl, a, b)


def _box4(v, s, out_rows):
    mid = out_rows + (2 * s) // 128 + 1
    x = v[0:mid] + _shifted(v, s, mid)
    return x[0:out_rows] + _shifted(x, 2 * s, out_rows)


def _fused_box(a_ref, b_ref, o_ref):
    u = jnp.concatenate([a_ref[...], b_ref[...]], axis=0)
    l1 = _BLK + (3 * _FACT[1]) // 128 + 3 + 5
    l2 = _BLK + (3 * _FACT[2]) // 128 + 3 + 2
    l3 = _BLK + 3
    x = _box4(u, _FACT[0], l1)
    x = _box4(x, _FACT[1], l2)
    x = _box4(x, _FACT[2], l3)
    o_ref[...] = _box4(x, 1, _BLK)


def _build_table(b_m):
    bp = jnp.pad(b_m, (0, _ROWS * 128 - _N)).reshape(_ROWS, 128)
    t = pl.pallas_call(
        _fused_box,
        grid=(_GRID,),
        in_specs=[
            pl.BlockSpec((_BLK, 128), lambda i: (i, 0)),
            pl.BlockSpec((_BLK, 128), lambda i: (i + 1, 0)),
        ],
        out_specs=pl.BlockSpec((_BLK, 128), lambda i: (i, 0)),
        out_shape=jax.ShapeDtypeStruct((_ROWS, 128), jnp.float32),
    )(bp, bp)
    return t.reshape(_ROWS * 128)


def _sc_lookup(x_flat, table, dw, mn):
    mesh = plsc.VectorSubcoreMesh(core_axis_name="c", subcore_axis_name="s")

    @functools.partial(
        pl.kernel,
        mesh=mesh,
        out_type=jax.ShapeDtypeStruct((_BATCH,), jnp.float32),
        scratch_types=[
            pltpu.VMEM((_BPW * 4,), jnp.float32),
            pltpu.VMEM((16,), jnp.float32),
            pltpu.VMEM((16,), jnp.float32),
            pltpu.VMEM((128,), jnp.int32),
            pltpu.VMEM((128,), jnp.int32),
            pltpu.VMEM((128,), jnp.int32),
            pltpu.VMEM((128,), jnp.int32),
            pltpu.VMEM((_BPW,), jnp.float32),
            pltpu.SemaphoreType.DMA,
        ],
    )
    def body(x_hbm, t_hbm, dw_hbm, mn_hbm, out_hbm,
             xbuf, dwv, mnv, r0, r1, r2, r3, ybuf, sem):
        wid = lax.axis_index("s") * 2 + lax.axis_index("c")
        base = wid * _BPW
        incopies = [pltpu.async_copy(x_hbm.at[pl.ds(d * _BATCH + base, _BPW)],
                                     xbuf.at[pl.ds(d * _BPW, _BPW)], sem)
                    for d in range(4)]
        incopies.append(pltpu.async_copy(dw_hbm, dwv.at[pl.ds(0, 4)], sem))
        incopies.append(pltpu.async_copy(mn_hbm, mnv.at[pl.ds(0, 4)], sem))
        for c in incopies:
            c.wait()
        rowrefs = (r0, r1, r2, r3)
        dwvec = dwv[...]
        mnvec = mnv[...]
        for g in range(_GROUPS):
            f = jnp.zeros((16,), jnp.int32)
            for d in range(4):
                xv = xbuf[pl.ds(d * _BPW + g * 16, 16)]
                a = ((xv - mnvec[d]) / dwvec[d]).astype(jnp.int32)
                f = f + a * _FACT[d]
            j, o = divmod(g, 8)
            rowrefs[j][pl.ds(o * 16, 16)] = f
        copies = [pltpu.async_copy(t_hbm.at[rowrefs[j]],
                                   ybuf.at[pl.ds(j * 128, 128)], sem)
                  for j in range(4)]
        for c in copies:
            c.wait()
        pltpu.sync_copy(ybuf, out_hbm.at[pl.ds(base, _BPW)])

    return body(x_flat, table, dw, mn)


def kernel(x, b_m, div_widths, min_dim_ranges, dim_order, dim_factors,
           bump_ind_offsets):
    table = _build_table(b_m)
    y = _sc_lookup(x.T.reshape(-1), table, div_widths, min_dim_ranges)
    return y.reshape(_BATCH, 1)
